# Initial kernel scaffold; baseline (speedup 1.0000x reference)
import functools

import jax
import jax.numpy as jnp
from jax import lax
from jax.experimental import pallas as pl
from jax.experimental.pallas import tpu as pltpu
from jax.experimental.pallas import tpu_sc as plsc

_N = 4096
_NB = 64
_NW = 32
_RPW = _N // _NW
_IV = _RPW // 16


def _sc_occupancy(xs, ys):
    mesh = plsc.VectorSubcoreMesh(core_axis_name="c", subcore_axis_name="s")

    @functools.partial(
        pl.kernel,
        out_type=jax.ShapeDtypeStruct((_N * _NB,), jnp.float32),
        mesh=mesh,
        scratch_types=[
            pltpu.VMEM((_N,), jnp.float32),
            pltpu.VMEM((_N,), jnp.float32),
            pltpu.VMEM((_RPW * _NB,), jnp.float32),
        ],
    )
    def occ_kernel(xs_hbm, ys_hbm, occ_hbm, xs_v, ys_v, occ_v):
        cid = lax.axis_index("c")
        sid = lax.axis_index("s")
        wid = sid * 2 + cid
        base = pl.multiple_of(wid * _RPW, _RPW)

        pltpu.sync_copy(xs_hbm, xs_v)
        pltpu.sync_copy(ys_hbm, ys_v)

        zero16 = jnp.zeros((16,), jnp.float32)

        def zbody(k, carry):
            occ_v[pl.ds(k * 16, 16)] = zero16
            return carry

        lax.fori_loop(0, _RPW * _NB // 16, zbody, 0)

        lane = lax.iota(jnp.int32, 16)
        ones = jnp.ones((16,), jnp.float32)

        cxs, cys, ibs = [], [], []
        for iv in range(_IV):
            i0 = pl.multiple_of(base + iv * 16, 16)
            cxs.append(xs_v[pl.ds(i0, 16)] - 11.0)
            cys.append(ys_v[pl.ds(i0, 16)] - 11.0)
            ibs.append((lane + iv * 16) * _NB - 63)

        def jbody(j, carry):
            xj = jnp.full((16,), xs_v[j], jnp.float32)
            yj = jnp.full((16,), ys_v[j], jnp.float32)
            for iv in range(_IV):
                rx = xj - cxs[iv]
                ry = yj - cys[iv]
                rx = jnp.minimum(jnp.maximum(rx, 7.0), 14.0)
                ry = jnp.minimum(jnp.maximum(ry, 7.0), 14.0)
                bx = rx.astype(jnp.int32)
                by = ry.astype(jnp.int32)
                idx = ibs[iv] + (bx * 8 + by)
                plsc.addupdate_scatter(occ_v, [idx], ones)
            return carry

        lax.fori_loop(0, _N, jbody, 0)

        pltpu.sync_copy(
            occ_v, occ_hbm.at[pl.ds(pl.multiple_of(base * _NB, 8), _RPW * _NB)]
        )

    return occ_kernel(xs, ys)


def _tc_linear(occ64, w64, b2):
    def mm_kernel(occ_ref, w_ref, b_ref, o_ref):
        o_ref[...] = (
            jnp.dot(occ_ref[...], w_ref[...], preferred_element_type=jnp.float32)
            + b_ref[...]
        )

    return pl.pallas_call(
        mm_kernel,
        grid=(8,),
        in_specs=[
            pl.BlockSpec((_N // 8, _NB), lambda i: (i, 0)),
            pl.BlockSpec((_NB, 128), lambda i: (0, 0)),
            pl.BlockSpec((1, 128), lambda i: (0, 0)),
        ],
        out_specs=pl.BlockSpec((_N // 8, 128), lambda i: (i, 0)),
        out_shape=jax.ShapeDtypeStruct((_N, 128), jnp.float32),
    )(occ64, w64, b2)


@jax.jit
def kernel(hidden_in, cell_in, obs, W, b):
    del hidden_in, cell_in
    xs = obs[:, 0] * 2.0
    ys = obs[:, 1] * 2.0

    occ64 = _sc_occupancy(xs, ys).reshape(_N, _NB)

    c36 = jnp.arange(36, dtype=jnp.int32)
    cols = (c36 // 6) * 8 + (c36 % 6) + 9
    w64 = jnp.zeros((_NB, 128), jnp.float32).at[cols].set(W.T)
    b2 = (b - W[:, 21])[None, :]

    return _tc_linear(occ64, w64, b2)


# speedup vs baseline: 104.2751x; 104.2751x over previous
import functools

import jax
import jax.numpy as jnp
from jax import lax
from jax.experimental import pallas as pl
from jax.experimental.pallas import tpu as pltpu
from jax.experimental.pallas import tpu_sc as plsc

_N = 4096
_NB = 64
_NW = 32
_RPW = _N // _NW
_IV = _RPW // 16


def _sc_occupancy(xs, ys):
    mesh = plsc.VectorSubcoreMesh(core_axis_name="c", subcore_axis_name="s")

    @functools.partial(
        pl.kernel,
        out_type=jax.ShapeDtypeStruct((_N * _NB,), jnp.float32),
        mesh=mesh,
        scratch_types=[
            pltpu.VMEM((_N,), jnp.float32),
            pltpu.VMEM((_N,), jnp.float32),
            pltpu.VMEM((_RPW * _NB,), jnp.float32),
        ],
        compiler_params=pltpu.CompilerParams(needs_layout_passes=False),
    )
    def occ_kernel(xs_hbm, ys_hbm, occ_hbm, xs_v, ys_v, occ_v):
        cid = lax.axis_index("c")
        sid = lax.axis_index("s")
        wid = sid * 2 + cid
        base = pl.multiple_of(wid * _RPW, _RPW)

        pltpu.sync_copy(xs_hbm, xs_v)
        pltpu.sync_copy(ys_hbm, ys_v)

        zero16 = jnp.zeros((16,), jnp.float32)

        def zbody(k, carry):
            occ_v[pl.ds(k * 16, 16)] = zero16
            return carry

        lax.fori_loop(0, _RPW * _NB // 16, zbody, 0)

        lane = lax.iota(jnp.int32, 16)
        ones = jnp.ones((16,), jnp.float32)

        cxs, cys, ibs = [], [], []
        for iv in range(_IV):
            i0 = pl.multiple_of(base + iv * 16, 16)
            cxs.append(xs_v[pl.ds(i0, 16)] - 11.0)
            cys.append(ys_v[pl.ds(i0, 16)] - 11.0)
            ibs.append((lane + iv * 16) * _NB - 63)

        def jbody(jc, carry):
            j0 = pl.multiple_of(jc * 16, 16)
            xchunk = xs_v[pl.ds(j0, 16)]
            ychunk = ys_v[pl.ds(j0, 16)]
            for jj in range(16):
                xj = jnp.full((16,), xchunk[jj], jnp.float32)
                yj = jnp.full((16,), ychunk[jj], jnp.float32)
                for iv in range(_IV):
                    rx = xj - cxs[iv]
                    ry = yj - cys[iv]
                    rx = jnp.minimum(jnp.maximum(rx, 7.0), 14.0)
                    ry = jnp.minimum(jnp.maximum(ry, 7.0), 14.0)
                    bx = rx.astype(jnp.int32)
                    by = ry.astype(jnp.int32)
                    idx = ibs[iv] + (bx * 8 + by)
                    plsc.addupdate_scatter(occ_v, [idx], ones)
            return carry

        lax.fori_loop(0, _N // 16, jbody, 0)

        pltpu.sync_copy(
            occ_v, occ_hbm.at[pl.ds(pl.multiple_of(base * _NB, 8), _RPW * _NB)]
        )

    return occ_kernel(xs, ys)


def _tc_linear(occ64, w64, b2):
    def mm_kernel(occ_ref, w_ref, b_ref, o_ref):
        o_ref[...] = (
            jnp.dot(occ_ref[...], w_ref[...], preferred_element_type=jnp.float32)
            + b_ref[...]
        )

    return pl.pallas_call(
        mm_kernel,
        grid=(8,),
        in_specs=[
            pl.BlockSpec((_N // 8, _NB), lambda i: (i, 0)),
            pl.BlockSpec((_NB, 128), lambda i: (0, 0)),
            pl.BlockSpec((1, 128), lambda i: (0, 0)),
        ],
        out_specs=pl.BlockSpec((_N // 8, 128), lambda i: (i, 0)),
        out_shape=jax.ShapeDtypeStruct((_N, 128), jnp.float32),
    )(occ64, w64, b2)


@jax.jit
def kernel(hidden_in, cell_in, obs, W, b):
    del hidden_in, cell_in
    xs = obs[:, 0] * 2.0
    ys = obs[:, 1] * 2.0

    occ64 = _sc_occupancy(xs, ys).reshape(_N, _NB)

    c36 = jnp.arange(36, dtype=jnp.int32)
    cols = (c36 // 6) * 8 + (c36 % 6) + 9
    w64 = jnp.zeros((_NB, 128), jnp.float32).at[cols].set(W.T)
    b2 = (b - W[:, 21])[None, :]

    return _tc_linear(occ64, w64, b2)


# speedup vs baseline: 131.8927x; 1.2649x over previous
import functools

import numpy as np

import jax
import jax.numpy as jnp
from jax import lax
from jax.experimental import pallas as pl
from jax.experimental.pallas import tpu as pltpu
from jax.experimental.pallas import tpu_sc as plsc

_N = 4096
_NB = 64
_NW = 32
_RPW = _N // _NW
_IV = _RPW // 16
_CLAMP_HI = float(np.nextafter(np.float32(16.0), np.float32(0.0)))


def _sc_occupancy(xs, ys):
    mesh = plsc.VectorSubcoreMesh(core_axis_name="c", subcore_axis_name="s")

    @functools.partial(
        pl.kernel,
        out_type=jax.ShapeDtypeStruct((_N * _NB,), jnp.float32),
        mesh=mesh,
        scratch_types=[
            pltpu.VMEM((_N,), jnp.float32),
            pltpu.VMEM((_N,), jnp.float32),
            pltpu.VMEM((_RPW * _NB,), jnp.float32),
        ],
        compiler_params=pltpu.CompilerParams(needs_layout_passes=False),
    )
    def occ_kernel(xs_hbm, ys_hbm, occ_hbm, xs_v, ys_v, occ_v):
        cid = lax.axis_index("c")
        sid = lax.axis_index("s")
        wid = sid * 2 + cid
        base = pl.multiple_of(wid * _RPW, _RPW)

        pltpu.sync_copy(xs_hbm, xs_v)
        pltpu.sync_copy(ys_hbm, ys_v)

        zero16 = jnp.zeros((16,), jnp.float32)

        def zbody(k, carry):
            occ_v[pl.ds(k * 16, 16)] = zero16
            return carry

        lax.fori_loop(0, _RPW * _NB // 16, zbody, 0)

        lane = lax.iota(jnp.int32, 16)
        ones = jnp.ones((16,), jnp.float32)

        cxs, cys, ibs = [], [], []
        for iv in range(_IV):
            i0 = pl.multiple_of(base + iv * 16, 16)
            cxs.append(xs_v[pl.ds(i0, 16)] - 12.0)
            cys.append(ys_v[pl.ds(i0, 16)] - 12.0)
            ibs.append((lane + iv * 16) * _NB - 0x410)

        def jbody(jc, carry):
            j0 = pl.multiple_of(jc * 16, 16)
            xchunk = xs_v[pl.ds(j0, 16)]
            ychunk = ys_v[pl.ds(j0, 16)]
            for jj in range(16):
                xj = jnp.full((16,), xchunk[jj], jnp.float32)
                yj = jnp.full((16,), ychunk[jj], jnp.float32)
                for iv in range(_IV):
                    rx = xj - cxs[iv]
                    ry = yj - cys[iv]
                    rx = jnp.minimum(jnp.maximum(rx, 8.0), _CLAMP_HI)
                    ry = jnp.minimum(jnp.maximum(ry, 8.0), _CLAMP_HI)
                    bxx = plsc.bitcast(rx, jnp.int32)
                    byy = plsc.bitcast(ry, jnp.int32)
                    col = lax.shift_right_logical(bxx, 17) & 0x38
                    idx = (ibs[iv] + col) + lax.shift_right_logical(byy, 20)
                    plsc.addupdate_scatter(occ_v, [idx], ones)
            return carry

        lax.fori_loop(0, _N // 16, jbody, 0)

        pltpu.sync_copy(
            occ_v, occ_hbm.at[pl.ds(pl.multiple_of(base * _NB, 8), _RPW * _NB)]
        )

    return occ_kernel(xs, ys)


def _tc_linear(occ64, w64, b2):
    def mm_kernel(occ_ref, w_ref, b_ref, o_ref):
        o_ref[...] = (
            jnp.dot(occ_ref[...], w_ref[...], preferred_element_type=jnp.float32)
            + b_ref[...]
        )

    return pl.pallas_call(
        mm_kernel,
        grid=(8,),
        in_specs=[
            pl.BlockSpec((_N // 8, _NB), lambda i: (i, 0)),
            pl.BlockSpec((_NB, 128), lambda i: (0, 0)),
            pl.BlockSpec((1, 128), lambda i: (0, 0)),
        ],
        out_specs=pl.BlockSpec((_N // 8, 128), lambda i: (i, 0)),
        out_shape=jax.ShapeDtypeStruct((_N, 128), jnp.float32),
    )(occ64, w64, b2)


@jax.jit
def kernel(hidden_in, cell_in, obs, W, b):
    del hidden_in, cell_in
    xs = obs[:, 0] * 2.0
    ys = obs[:, 1] * 2.0

    occ64 = _sc_occupancy(xs, ys).reshape(_N, _NB)

    c36 = jnp.arange(36, dtype=jnp.int32)
    cols = (c36 // 6) * 8 + (c36 % 6) + 9
    w64 = jnp.zeros((_NB, 128), jnp.float32).at[cols].set(W.T)
    b2 = (b - W[:, 21])[None, :]

    return _tc_linear(occ64, w64, b2)
